# Initial kernel scaffold; baseline (speedup 1.0000x reference)
import jax
import jax.numpy as jnp
from jax.experimental import pallas as pl


def kernel(x, edge_index, batch, W1, b1, W2, b2, Wc1, bc1, Wc2, bc2):
    raise NotImplementedError("write your pallas kernel here")


# speedup vs baseline: 18.6654x; 18.6654x over previous
import functools

import jax
import jax.numpy as jnp
from jax import lax
from jax.experimental import pallas as pl
from jax.experimental.pallas import tpu as pltpu
from jax.experimental.pallas import tpu_sc as plsc

N = 10000
E = 320000
D = 128
H = 128
OUT = 10
G = 64

NC = 2
NS = 16
NW = NC * NS
EW = E // NW
K = 100
C = EW // K
STRIPE = 1000
NDRAIN = N // STRIPE

_MESH = plsc.VectorSubcoreMesh(core_axis_name="c", subcore_axis_name="s")


def _deg_body(dst_hbm, ones_hbm, zeros_hbm, out_hbm, dstv, onesv, acc):
    cid = lax.axis_index("c")
    sid = lax.axis_index("s")
    wid = cid * NS + sid

    @pl.when(sid < NDRAIN)
    def _():
        pltpu.sync_copy(zeros_hbm, acc.at[pl.ds(sid * STRIPE, STRIPE)])

    pltpu.sync_copy(ones_hbm, onesv)
    pltpu.sync_copy(dst_hbm.at[wid], dstv)
    plsc.subcore_barrier()

    def step(j, carry):
        pltpu.sync_copy(onesv, acc.at[dstv.at[j]], add=True)
        return carry

    lax.fori_loop(0, C, step, 0)
    plsc.subcore_barrier()

    @pl.when(sid < NDRAIN)
    def _():
        pltpu.sync_copy(acc.at[pl.ds(sid * STRIPE, STRIPE)],
                        out_hbm.at[cid, pl.ds(sid * STRIPE, STRIPE)])


_deg_kernel = pl.kernel(
    _deg_body,
    out_type=jax.ShapeDtypeStruct((NC, N, H), jnp.float32),
    mesh=_MESH,
    scratch_types=[
        pltpu.VMEM((C, K), jnp.int32),
        pltpu.VMEM((K, H), jnp.float32),
        pltpu.VMEM_SHARED((N, H), jnp.float32),
    ],
)


def _scatter_body(hn_hbm, src_hbm, dst_hbm, zeros_hbm, out_hbm,
                  srcv, dstv, rows, acc, sem):
    cid = lax.axis_index("c")
    sid = lax.axis_index("s")
    wid = cid * NS + sid

    @pl.when(sid < NDRAIN)
    def _():
        pltpu.sync_copy(zeros_hbm, acc.at[pl.ds(sid * STRIPE, STRIPE)])

    pltpu.sync_copy(src_hbm.at[wid], srcv)
    pltpu.sync_copy(dst_hbm.at[wid], dstv)
    plsc.subcore_barrier()

    def step(j, carry):
        pltpu.async_copy(hn_hbm.at[srcv.at[j]], rows, sem).wait()
        pltpu.sync_copy(rows, acc.at[dstv.at[j]], add=True)
        return carry

    lax.fori_loop(0, C, step, 0)
    plsc.subcore_barrier()

    @pl.when(sid < NDRAIN)
    def _():
        pltpu.sync_copy(acc.at[pl.ds(sid * STRIPE, STRIPE)],
                        out_hbm.at[cid, pl.ds(sid * STRIPE, STRIPE)])


_scatter_kernel = pl.kernel(
    _scatter_body,
    out_type=jax.ShapeDtypeStruct((NC, N, H), jnp.float32),
    mesh=_MESH,
    scratch_types=[
        pltpu.VMEM((C, K), jnp.int32),
        pltpu.VMEM((C, K), jnp.int32),
        pltpu.VMEM((K, H), jnp.float32),
        pltpu.VMEM_SHARED((N, H), jnp.float32),
        pltpu.SemaphoreType.DMA,
    ],
)


_BR = 2000
_NB = N // _BR


def _tc1_body(x_ref, w1_ref, p0_ref, p1_ref, hn_ref, dinv_ref):
    deg = p0_ref[...] + p1_ref[...] + 1.0
    dinv = lax.rsqrt(deg)
    y = lax.dot_general(x_ref[...], w1_ref[...],
                        (((1,), (1,)), ((), ())),
                        preferred_element_type=jnp.float32)
    hn_ref[...] = y * dinv
    dinv_ref[...] = dinv


def _tc2_body(s0_ref, s1_ref, hn_ref, dinv_ref, b1_ref, w2_ref, out_ref):
    dinv = dinv_ref[...]
    h1 = dinv * (s0_ref[...] + s1_ref[...] + hn_ref[...]) + b1_ref[...]
    h1 = jnp.maximum(h1, 0.0)
    y = lax.dot_general(h1, w2_ref[...],
                        (((1,), (1,)), ((), ())),
                        preferred_element_type=jnp.float32)
    out_ref[...] = y * dinv


def _tc3_body(s0_ref, s1_ref, hn_ref, dinv_ref, b2_ref, batch_ref,
              wc1_ref, bc1_ref, wc2_ref, bc2_ref, out_ref):
    h2 = dinv_ref[...] * (s0_ref[...] + s1_ref[...] + hn_ref[...]) + b2_ref[...]
    h2 = jnp.maximum(h2, 0.0)
    gids = lax.broadcasted_iota(jnp.int32, (G, N), 0)
    m = (gids == batch_ref[...]).astype(jnp.float32)
    sums = lax.dot_general(m, h2, (((1,), (0,)), ((), ())),
                           preferred_element_type=jnp.float32)
    counts = jnp.sum(m, axis=1, keepdims=True)
    pooled = sums / jnp.maximum(counts, 1.0)
    z = lax.dot_general(pooled, wc1_ref[...], (((1,), (1,)), ((), ())),
                        preferred_element_type=jnp.float32) + bc1_ref[...]
    z = jnp.maximum(z, 0.0)
    out_ref[...] = lax.dot_general(z, wc2_ref[...], (((1,), (1,)), ((), ())),
                                   preferred_element_type=jnp.float32) + bc2_ref[...]


def _rows_spec(w):
    return pl.BlockSpec((_BR, w), lambda i: (i, 0))


def _full_spec(shape):
    return pl.BlockSpec(shape, lambda i: (0,) * len(shape))


_tc1 = pl.pallas_call(
    _tc1_body,
    grid=(_NB,),
    in_specs=[_rows_spec(D), _full_spec((H, D)), _rows_spec(1), _rows_spec(1)],
    out_specs=[_rows_spec(H), _rows_spec(1)],
    out_shape=[jax.ShapeDtypeStruct((N, H), jnp.float32),
               jax.ShapeDtypeStruct((N, 1), jnp.float32)],
)

_tc2 = pl.pallas_call(
    _tc2_body,
    grid=(_NB,),
    in_specs=[_rows_spec(H), _rows_spec(H), _rows_spec(H), _rows_spec(1),
              _full_spec((1, H)), _full_spec((H, H))],
    out_specs=_rows_spec(H),
    out_shape=jax.ShapeDtypeStruct((N, H), jnp.float32),
)

_tc3 = pl.pallas_call(
    _tc3_body,
    out_shape=jax.ShapeDtypeStruct((G, OUT), jnp.float32),
)


def kernel(x, edge_index, batch, W1, b1, W2, b2, Wc1, bc1, Wc2, bc2):
    src3 = edge_index[0].reshape(NW, C, K)
    dst3 = edge_index[1].reshape(NW, C, K)
    ones_kH = jnp.ones((K, H), jnp.float32)
    zerosH = jnp.zeros((STRIPE, H), jnp.float32)

    dd = _deg_kernel(dst3, ones_kH, zerosH)
    p0 = dd[0, :, 0:1]
    p1 = dd[1, :, 0:1]

    hn1, dinv = _tc1(x, W1, p0, p1)

    s1 = _scatter_kernel(hn1, src3, dst3, zerosH)
    hn2 = _tc2(s1[0], s1[1], hn1, dinv, b1.reshape(1, H), W2)

    s2 = _scatter_kernel(hn2, src3, dst3, zerosH)
    out = _tc3(s2[0], s2[1], hn2, dinv, b2.reshape(1, H),
               batch.reshape(1, N), Wc1, bc1.reshape(1, H // 2),
               Wc2, bc2.reshape(1, OUT))
    return out


# speedup vs baseline: 26.5973x; 1.4249x over previous
import functools

import jax
import jax.numpy as jnp
from jax import lax
from jax.experimental import pallas as pl
from jax.experimental.pallas import tpu as pltpu
from jax.experimental.pallas import tpu_sc as plsc

N = 10000
E = 320000
D = 128
H = 128
OUT = 10
G = 64

NC = 2
NS = 16
NW = NC * NS
EW = E // NW
K = 125
C = EW // K
P = 2
CP = C // P
STRIPE = 1000
NDRAIN = N // STRIPE

_MESH = plsc.VectorSubcoreMesh(core_axis_name="c", subcore_axis_name="s")


def _deg_body(dst_hbm, ones_hbm, zeros_hbm, out_hbm, dstv, onesv, acc):
    cid = lax.axis_index("c")
    sid = lax.axis_index("s")
    wid = cid * NS + sid

    @pl.when(sid < NDRAIN)
    def _():
        pltpu.sync_copy(zeros_hbm, acc.at[pl.ds(sid * STRIPE, STRIPE)])

    pltpu.sync_copy(ones_hbm, onesv)
    pltpu.sync_copy(dst_hbm.at[wid], dstv)
    plsc.subcore_barrier()

    def step(j, carry):
        pltpu.sync_copy(onesv, acc.at[dstv.at[j]], add=True)
        return carry

    lax.fori_loop(0, C, step, 0)
    plsc.subcore_barrier()

    @pl.when(sid < NDRAIN)
    def _():
        pltpu.sync_copy(acc.at[pl.ds(sid * STRIPE, STRIPE)],
                        out_hbm.at[cid, pl.ds(sid * STRIPE, STRIPE)])


_deg_kernel = pl.kernel(
    _deg_body,
    out_type=jax.ShapeDtypeStruct((NC, N, H), jnp.float32),
    mesh=_MESH,
    scratch_types=[
        pltpu.VMEM((C, K), jnp.int32),
        pltpu.VMEM((K, H), jnp.float32),
        pltpu.VMEM_SHARED((N, H), jnp.float32),
    ],
)


def _scatter_body(hn_hbm, src_hbm, dst_hbm, zeros_hbm, out_hbm,
                  srcv, dstv, rows0, rows1, acc, sem0, sem1):
    cid = lax.axis_index("c")
    sid = lax.axis_index("s")
    wid = cid * NS + sid

    @pl.when(sid < NDRAIN)
    def _():
        pltpu.sync_copy(zeros_hbm, acc.at[pl.ds(sid * STRIPE, STRIPE)])

    plsc.subcore_barrier()

    for p in range(P):
        pltpu.sync_copy(src_hbm.at[wid, pl.ds(p * CP, CP)], srcv)
        pltpu.sync_copy(dst_hbm.at[wid, pl.ds(p * CP, CP)], dstv)
        pltpu.async_copy(hn_hbm.at[srcv.at[0]], rows0, sem0)

        def step(i, carry):
            j0 = 2 * i
            j1 = 2 * i + 1
            pltpu.async_copy(hn_hbm.at[srcv.at[j1]], rows1, sem1)
            pltpu.make_async_copy(hn_hbm.at[srcv.at[j0]], rows0, sem0).wait()
            pltpu.sync_copy(rows0, acc.at[dstv.at[j0]], add=True)

            @pl.when(i + 1 < CP // 2)
            def _():
                pltpu.async_copy(hn_hbm.at[srcv.at[j0 + 2]], rows0, sem0)

            pltpu.make_async_copy(hn_hbm.at[srcv.at[j1]], rows1, sem1).wait()
            pltpu.sync_copy(rows1, acc.at[dstv.at[j1]], add=True)
            return carry

        lax.fori_loop(0, CP // 2, step, 0)

    plsc.subcore_barrier()

    @pl.when(sid < NDRAIN)
    def _():
        pltpu.sync_copy(acc.at[pl.ds(sid * STRIPE, STRIPE)],
                        out_hbm.at[cid, pl.ds(sid * STRIPE, STRIPE)])


_scatter_kernel = pl.kernel(
    _scatter_body,
    out_type=jax.ShapeDtypeStruct((NC, N, H), jnp.float32),
    mesh=_MESH,
    scratch_types=[
        pltpu.VMEM((CP, K), jnp.int32),
        pltpu.VMEM((CP, K), jnp.int32),
        pltpu.VMEM((K, H), jnp.float32),
        pltpu.VMEM((K, H), jnp.float32),
        pltpu.VMEM_SHARED((N, H), jnp.float32),
        pltpu.SemaphoreType.DMA,
        pltpu.SemaphoreType.DMA,
    ],
)


_BR = 2000
_NB = N // _BR


def _tc1_body(x_ref, w1_ref, p0_ref, p1_ref, hn_ref, dinv_ref):
    deg = p0_ref[...] + p1_ref[...] + 1.0
    dinv = lax.rsqrt(deg)
    y = lax.dot_general(x_ref[...], w1_ref[...],
                        (((1,), (1,)), ((), ())),
                        preferred_element_type=jnp.float32)
    hn_ref[...] = y * dinv
    dinv_ref[...] = dinv


def _tc2_body(s0_ref, s1_ref, hn_ref, dinv_ref, b1_ref, w2_ref, out_ref):
    dinv = dinv_ref[...]
    h1 = dinv * (s0_ref[...] + s1_ref[...] + hn_ref[...]) + b1_ref[...]
    h1 = jnp.maximum(h1, 0.0)
    y = lax.dot_general(h1, w2_ref[...],
                        (((1,), (1,)), ((), ())),
                        preferred_element_type=jnp.float32)
    out_ref[...] = y * dinv


def _tc3_body(s0_ref, s1_ref, hn_ref, dinv_ref, b2_ref, batch_ref,
              wc1_ref, bc1_ref, wc2_ref, bc2_ref, out_ref):
    h2 = dinv_ref[...] * (s0_ref[...] + s1_ref[...] + hn_ref[...]) + b2_ref[...]
    h2 = jnp.maximum(h2, 0.0)
    gids = lax.broadcasted_iota(jnp.int32, (G, N), 0)
    m = (gids == batch_ref[...]).astype(jnp.float32)
    sums = lax.dot_general(m, h2, (((1,), (0,)), ((), ())),
                           preferred_element_type=jnp.float32)
    counts = jnp.sum(m, axis=1, keepdims=True)
    pooled = sums / jnp.maximum(counts, 1.0)
    z = lax.dot_general(pooled, wc1_ref[...], (((1,), (1,)), ((), ())),
                        preferred_element_type=jnp.float32) + bc1_ref[...]
    z = jnp.maximum(z, 0.0)
    out_ref[...] = lax.dot_general(z, wc2_ref[...], (((1,), (1,)), ((), ())),
                                   preferred_element_type=jnp.float32) + bc2_ref[...]


def _rows_spec(w):
    return pl.BlockSpec((_BR, w), lambda i: (i, 0))


def _full_spec(shape):
    return pl.BlockSpec(shape, lambda i: (0,) * len(shape))


_tc1 = pl.pallas_call(
    _tc1_body,
    grid=(_NB,),
    in_specs=[_rows_spec(D), _full_spec((H, D)), _rows_spec(1), _rows_spec(1)],
    out_specs=[_rows_spec(H), _rows_spec(1)],
    out_shape=[jax.ShapeDtypeStruct((N, H), jnp.float32),
               jax.ShapeDtypeStruct((N, 1), jnp.float32)],
)

_tc2 = pl.pallas_call(
    _tc2_body,
    grid=(_NB,),
    in_specs=[_rows_spec(H), _rows_spec(H), _rows_spec(H), _rows_spec(1),
              _full_spec((1, H)), _full_spec((H, H))],
    out_specs=_rows_spec(H),
    out_shape=jax.ShapeDtypeStruct((N, H), jnp.float32),
)

_tc3 = pl.pallas_call(
    _tc3_body,
    out_shape=jax.ShapeDtypeStruct((G, OUT), jnp.float32),
)


def kernel(x, edge_index, batch, W1, b1, W2, b2, Wc1, bc1, Wc2, bc2):
    src3 = edge_index[0].reshape(NW, C, K)
    dst3 = edge_index[1].reshape(NW, C, K)
    ones_kH = jnp.ones((K, H), jnp.float32)
    zerosH = jnp.zeros((STRIPE, H), jnp.float32)

    dd = _deg_kernel(dst3, ones_kH, zerosH)
    p0 = dd[0, :, 0:1]
    p1 = dd[1, :, 0:1]

    hn1, dinv = _tc1(x, W1, p0, p1)

    s1 = _scatter_kernel(hn1, src3, dst3, zerosH)
    hn2 = _tc2(s1[0], s1[1], hn1, dinv, b1.reshape(1, H), W2)

    s2 = _scatter_kernel(hn2, src3, dst3, zerosH)
    out = _tc3(s2[0], s2[1], hn2, dinv, b2.reshape(1, H),
               batch.reshape(1, N), Wc1, bc1.reshape(1, H // 2),
               Wc2, bc2.reshape(1, OUT))
    return out


# speedup vs baseline: 31.4915x; 1.1840x over previous
import jax
import jax.numpy as jnp
from jax import lax
from jax.experimental import pallas as pl
from jax.experimental.pallas import tpu as pltpu
from jax.experimental.pallas import tpu_sc as plsc

N = 10000
E = 320000
D = 128
H = 128
OUT = 10
G = 64

NC = 2
NS = 16
NW = NC * NS
EW = E // NW
K = 125
C = EW // K
P = 2
CP = C // P
STRIPE = 1000
NDRAIN = N // STRIPE

_MESH = plsc.VectorSubcoreMesh(core_axis_name="c", subcore_axis_name="s")


RPT = (E // NW) // 128 + 1
HR = N // 128 + 3


def _deg_body(dst_hbm, idx_hbm, zeros_hbm, out_hbm, dstv, hist, idxv, acc):
    cid = lax.axis_index("c")
    sid = lax.axis_index("s")
    wid = cid * NS + sid
    pltpu.sync_copy(zeros_hbm, hist)

    @pl.when(sid == 0)
    def _():
        pltpu.sync_copy(zeros_hbm, acc)

    pltpu.sync_copy(idx_hbm, idxv)
    pltpu.sync_copy(dst_hbm.at[wid], dstv)

    def step(r, carry):
        for k in range(8):
            v = dstv[r, pl.ds(16 * k, 16)]
            row = lax.shift_right_logical(v, 7)
            col = lax.bitwise_and(v, 127)
            plsc.addupdate_scatter(hist, [row, col],
                                   jnp.ones((16,), jnp.float32))
        return carry

    lax.fori_loop(0, RPT, step, 0)
    plsc.subcore_barrier()
    pltpu.sync_copy(hist, acc.at[idxv.at[0]], add=True)
    plsc.subcore_barrier()

    @pl.when(sid == 0)
    def _():
        pltpu.sync_copy(acc, out_hbm.at[cid])


_deg_kernel = pl.kernel(
    _deg_body,
    out_type=jax.ShapeDtypeStruct((NC, HR, 128), jnp.float32),
    mesh=_MESH,
    compiler_params=pltpu.CompilerParams(needs_layout_passes=False),
    scratch_types=[
        pltpu.VMEM((RPT, 128), jnp.int32),
        pltpu.VMEM((HR, 128), jnp.float32),
        pltpu.VMEM((1, HR), jnp.int32),
        pltpu.VMEM_SHARED((HR, 128), jnp.float32),
    ],
)


def _scatter_body(hn_hbm, src_hbm, dst_hbm, zeros_hbm, out_hbm,
                  srcv, dstv, rows0, rows1, acc, sem0, sem1):
    cid = lax.axis_index("c")
    sid = lax.axis_index("s")
    wid = cid * NS + sid

    @pl.when((sid < NDRAIN) & (cid == 0))
    def _():
        pltpu.sync_copy(hn_hbm.at[pl.ds(sid * STRIPE, STRIPE)],
                        acc.at[pl.ds(sid * STRIPE, STRIPE)])

    @pl.when((sid < NDRAIN) & (cid == 1))
    def _():
        pltpu.sync_copy(zeros_hbm, acc.at[pl.ds(sid * STRIPE, STRIPE)])

    for p in range(P):
        pltpu.sync_copy(src_hbm.at[wid, pl.ds(p * CP, CP)], srcv)
        pltpu.sync_copy(dst_hbm.at[wid, pl.ds(p * CP, CP)], dstv)
        pltpu.async_copy(hn_hbm.at[srcv.at[0]], rows0, sem0)
        if p == 0:
            plsc.subcore_barrier()

        def step(i, carry):
            j0 = 2 * i
            j1 = 2 * i + 1
            pltpu.async_copy(hn_hbm.at[srcv.at[j1]], rows1, sem1)
            pltpu.make_async_copy(hn_hbm.at[srcv.at[j0]], rows0, sem0).wait()
            pltpu.sync_copy(rows0, acc.at[dstv.at[j0]], add=True)

            @pl.when(i + 1 < CP // 2)
            def _():
                pltpu.async_copy(hn_hbm.at[srcv.at[j0 + 2]], rows0, sem0)

            pltpu.make_async_copy(hn_hbm.at[srcv.at[j1]], rows1, sem1).wait()
            pltpu.sync_copy(rows1, acc.at[dstv.at[j1]], add=True)
            return carry

        lax.fori_loop(0, CP // 2, step, 0)

    plsc.subcore_barrier()

    @pl.when(sid < NDRAIN)
    def _():
        pltpu.sync_copy(acc.at[pl.ds(sid * STRIPE, STRIPE)],
                        out_hbm.at[cid, pl.ds(sid * STRIPE, STRIPE)])


_scatter_kernel = pl.kernel(
    _scatter_body,
    out_type=jax.ShapeDtypeStruct((NC, N, H), jnp.float32),
    mesh=_MESH,
    scratch_types=[
        pltpu.VMEM((CP, K), jnp.int32),
        pltpu.VMEM((CP, K), jnp.int32),
        pltpu.VMEM((K, H), jnp.float32),
        pltpu.VMEM((K, H), jnp.float32),
        pltpu.VMEM_SHARED((N, H), jnp.float32),
        pltpu.SemaphoreType.DMA,
        pltpu.SemaphoreType.DMA,
    ],
)


_BR = 2000
_NB = N // _BR


def _tc1_body(x_ref, w1_ref, p0_ref, p1_ref, hn_ref, dinv_ref):
    deg = p0_ref[...] + p1_ref[...] + 1.0
    dinv = lax.rsqrt(deg)
    y = lax.dot_general(x_ref[...], w1_ref[...],
                        (((1,), (1,)), ((), ())),
                        preferred_element_type=jnp.float32)
    hn_ref[...] = y * dinv
    dinv_ref[...] = dinv


def _tc2_body(s0_ref, s1_ref, dinv_ref, b1_ref, w2_ref, out_ref):
    dinv = dinv_ref[...]
    h1 = dinv * (s0_ref[...] + s1_ref[...]) + b1_ref[...]
    h1 = jnp.maximum(h1, 0.0)
    y = lax.dot_general(h1, w2_ref[...],
                        (((1,), (1,)), ((), ())),
                        preferred_element_type=jnp.float32)
    out_ref[...] = y * dinv


def _tc3_body(s0_ref, s1_ref, dinv_ref, b2_ref, batch_ref,
              wc1_ref, bc1_ref, wc2_ref, bc2_ref, out_ref):
    h2 = dinv_ref[...] * (s0_ref[...] + s1_ref[...]) + b2_ref[...]
    h2 = jnp.maximum(h2, 0.0)
    gids = lax.broadcasted_iota(jnp.int32, (G, N), 0)
    m = (gids == batch_ref[...]).astype(jnp.float32)
    sums = lax.dot_general(m, h2, (((1,), (0,)), ((), ())),
                           preferred_element_type=jnp.float32)
    counts = jnp.sum(m, axis=1, keepdims=True)
    pooled = sums / jnp.maximum(counts, 1.0)
    z = lax.dot_general(pooled, wc1_ref[...], (((1,), (1,)), ((), ())),
                        preferred_element_type=jnp.float32) + bc1_ref[...]
    z = jnp.maximum(z, 0.0)
    out_ref[...] = lax.dot_general(z, wc2_ref[...], (((1,), (1,)), ((), ())),
                                   preferred_element_type=jnp.float32) + bc2_ref[...]


def _rows_spec(w):
    return pl.BlockSpec((_BR, w), lambda i: (i, 0))


def _full_spec(shape):
    return pl.BlockSpec(shape, lambda i: (0,) * len(shape))


_tc1 = pl.pallas_call(
    _tc1_body,
    grid=(_NB,),
    in_specs=[_rows_spec(D), _full_spec((H, D)), _rows_spec(1), _rows_spec(1)],
    out_specs=[_rows_spec(H), _rows_spec(1)],
    out_shape=[jax.ShapeDtypeStruct((N, H), jnp.float32),
               jax.ShapeDtypeStruct((N, 1), jnp.float32)],
)

_tc2 = pl.pallas_call(
    _tc2_body,
    grid=(_NB,),
    in_specs=[_rows_spec(H), _rows_spec(H), _rows_spec(1),
              _full_spec((1, H)), _full_spec((H, H))],
    out_specs=_rows_spec(H),
    out_shape=jax.ShapeDtypeStruct((N, H), jnp.float32),
)

_tc3 = pl.pallas_call(
    _tc3_body,
    out_shape=jax.ShapeDtypeStruct((G, OUT), jnp.float32),
)


def kernel(x, edge_index, batch, W1, b1, W2, b2, Wc1, bc1, Wc2, bc2):
    src3 = edge_index[0].reshape(NW, C, K)
    dst3 = edge_index[1].reshape(NW, C, K)
    zerosH = jnp.zeros((STRIPE, H), jnp.float32)

    pad = jnp.full((NW * RPT * 128 - E,), N, jnp.int32)
    dstp = jnp.concatenate([edge_index[1], pad]).reshape(NW, RPT, 128)
    idx_hr = jnp.arange(HR, dtype=jnp.int32).reshape(1, HR)
    zeros_hr = jnp.zeros((HR, 128), jnp.float32)

    dd = _deg_kernel(dstp, idx_hr, zeros_hr)
    p0 = dd[0].reshape(-1)[:N, None]
    p1 = dd[1].reshape(-1)[:N, None]

    hn1, dinv = _tc1(x, W1, p0, p1)

    s1 = _scatter_kernel(hn1, src3, dst3, zerosH)
    hn2 = _tc2(s1[0], s1[1], dinv, b1.reshape(1, H), W2)

    s2 = _scatter_kernel(hn2, src3, dst3, zerosH)
    out = _tc3(s2[0], s2[1], dinv, b2.reshape(1, H),
               batch.reshape(1, N), Wc1, bc1.reshape(1, H // 2),
               Wc2, bc2.reshape(1, OUT))
    return out
